# Initial kernel scaffold; baseline (speedup 1.0000x reference)
import jax
import jax.numpy as jnp
from jax.experimental import pallas as pl


def kernel(x, edge_index, W1_self, W1_neigh, b1, W2_self, W2_neigh, b2):
    raise NotImplementedError("write your pallas kernel here")


# speedup vs baseline: 4.5509x; 4.5509x over previous
import functools

import jax
import jax.numpy as jnp
from jax import lax
from jax.experimental import pallas as pl
from jax.experimental.pallas import tpu as pltpu
from jax.experimental.pallas import tpu_sc as plsc

N = 10000
D = 128
E = 320000

NC = 2
NS = 16
DHALF = D // NC
EPT = E // NS
CH = 80
NCHUNK = EPT // CH
NPAD = 10240
RPT = NPAD // NS
DEGW = 16

_f32 = jnp.float32


def _sc_body(plo_hbm, phi_hbm, src_hbm, dst_hbm, agglo_out, agghi_out, deg_out,
             sidx, didx, rows, zbuf, ones_v, zb16, agg_sh, deg_sh, sem):
    c = lax.axis_index("c")
    s = lax.axis_index("s")

    zero16 = jnp.zeros((16,), _f32)

    def zrow(i, carry):
        for k in range(DHALF // 16):
            zbuf[i, pl.ds(16 * k, 16)] = zero16
        return carry

    lax.fori_loop(0, 128, zrow, 0)
    for k in range(RPT // 128):
        pltpu.sync_copy(zbuf, agg_sh.at[pl.ds(s * RPT + k * 128, 128)])

    @pl.when(c == 0)
    def _():
        one16 = jnp.ones((16,), _f32)

        def zrow16(i, carry):
            zb16[i, :] = zero16
            return carry

        lax.fori_loop(0, 128, zrow16, 0)
        for k in range(RPT // 128):
            pltpu.sync_copy(zb16, deg_sh.at[pl.ds(s * RPT + k * 128, 128)])

        def orow(i, carry):
            ones_v[i, :] = one16
            return carry

        lax.fori_loop(0, CH, orow, 0)

    plsc.subcore_barrier()

    pltpu.sync_copy(src_hbm.at[s], sidx)
    pltpu.sync_copy(dst_hbm.at[s], didx)

    def body(j, carry):
        @pl.when(c == 0)
        def _():
            pltpu.async_copy(plo_hbm.at[sidx.at[j]], rows, sem).wait()

        @pl.when(c == 1)
        def _():
            pltpu.async_copy(phi_hbm.at[sidx.at[j]], rows, sem).wait()

        pltpu.sync_copy(rows, agg_sh.at[didx.at[j]], add=True)

        @pl.when(c == 0)
        def _():
            pltpu.sync_copy(ones_v, deg_sh.at[didx.at[j]], add=True)

        return carry

    lax.fori_loop(0, NCHUNK, body, 0)

    plsc.subcore_barrier()

    my_rows = agg_sh.at[pl.ds(s * RPT, RPT)]

    @pl.when(c == 0)
    def _():
        pltpu.sync_copy(my_rows, agglo_out.at[pl.ds(s * RPT, RPT)])
        pltpu.sync_copy(deg_sh.at[pl.ds(s * RPT, RPT)],
                        deg_out.at[pl.ds(s * RPT, RPT)])

    @pl.when(c == 1)
    def _():
        pltpu.sync_copy(my_rows, agghi_out.at[pl.ds(s * RPT, RPT)])


@functools.lru_cache(maxsize=None)
def _make_sc():
    out_type = [
        jax.ShapeDtypeStruct((NPAD, DHALF), _f32),
        jax.ShapeDtypeStruct((NPAD, DHALF), _f32),
        jax.ShapeDtypeStruct((NPAD, DEGW), _f32),
    ]
    scratch = [
        pltpu.VMEM((NCHUNK, CH), jnp.int32),
        pltpu.VMEM((NCHUNK, CH), jnp.int32),
        pltpu.VMEM((CH, DHALF), _f32),
        pltpu.VMEM((128, DHALF), _f32),
        pltpu.VMEM((CH, DEGW), _f32),
        pltpu.VMEM((128, DEGW), _f32),
        pltpu.VMEM_SHARED((NPAD, DHALF), _f32),
        pltpu.VMEM_SHARED((NPAD, DEGW), _f32),
        pltpu.SemaphoreType.DMA,
    ]
    mesh = plsc.VectorSubcoreMesh(core_axis_name="c", subcore_axis_name="s")
    return pl.kernel(
        _sc_body,
        out_type=out_type,
        mesh=mesh,
        scratch_types=scratch,
        compiler_params=pltpu.CompilerParams(use_tc_tiling_on_sc=False),
        name="sage_sc_agg",
    )


_BM = 256
_GRID = (N + _BM - 1) // _BM

_dot = functools.partial(jnp.dot, preferred_element_type=_f32,
                         precision=jax.lax.Precision.HIGHEST)


def _tc1_body(x_ref, wn_ref, ws_ref, plo_ref, phi_ref, s_ref):
    xb = x_ref[...]
    p = _dot(xb, wn_ref[...])
    plo_ref[...] = p[:, :DHALF]
    phi_ref[...] = p[:, DHALF:]
    s_ref[...] = _dot(xb, ws_ref[...])


def _tc2_body(s1_ref, agglo_ref, agghi_ref, deg_ref, b_ref, wn_ref, ws_ref,
              plo_ref, phi_ref, s2_ref):
    deg = jnp.maximum(deg_ref[:, 0:1], 1.0)
    agg = jnp.concatenate([agglo_ref[...], agghi_ref[...]], axis=1)
    h = jnp.maximum(s1_ref[...] + agg / deg + b_ref[...], 0.0)
    p = _dot(h, wn_ref[...])
    plo_ref[...] = p[:, :DHALF]
    phi_ref[...] = p[:, DHALF:]
    s2_ref[...] = _dot(h, ws_ref[...])


def _tc3_body(s2_ref, agglo_ref, agghi_ref, deg_ref, b_ref, o_ref):
    deg = jnp.maximum(deg_ref[:, 0:1], 1.0)
    agg = jnp.concatenate([agglo_ref[...], agghi_ref[...]], axis=1)
    o_ref[...] = s2_ref[...] + agg / deg + b_ref[...]


_row_spec = pl.BlockSpec((_BM, D), lambda i: (i, 0))
_half_spec = pl.BlockSpec((_BM, DHALF), lambda i: (i, 0))
_w_spec = pl.BlockSpec((D, D), lambda i: (0, 0))
_b_spec = pl.BlockSpec((1, D), lambda i: (0, 0))
_deg_spec = pl.BlockSpec((_BM, DEGW), lambda i: (i, 0))
_row_out = jax.ShapeDtypeStruct((N, D), _f32)
_half_out = jax.ShapeDtypeStruct((N, DHALF), _f32)

_tc1 = pl.pallas_call(
    _tc1_body,
    grid=(_GRID,),
    in_specs=[_row_spec, _w_spec, _w_spec],
    out_specs=[_half_spec, _half_spec, _row_spec],
    out_shape=[_half_out, _half_out, _row_out],
)

_tc2 = pl.pallas_call(
    _tc2_body,
    grid=(_GRID,),
    in_specs=[_row_spec, _half_spec, _half_spec, _deg_spec, _b_spec,
              _w_spec, _w_spec],
    out_specs=[_half_spec, _half_spec, _row_spec],
    out_shape=[_half_out, _half_out, _row_out],
)

_tc3 = pl.pallas_call(
    _tc3_body,
    grid=(_GRID,),
    in_specs=[_row_spec, _half_spec, _half_spec, _deg_spec, _b_spec],
    out_specs=_row_spec,
    out_shape=_row_out,
)


def kernel(x, edge_index, W1_self, W1_neigh, b1, W2_self, W2_neigh, b2):
    src = edge_index[0].reshape(NS, NCHUNK, CH)
    dst = edge_index[1].reshape(NS, NCHUNK, CH)
    b1r = b1.reshape(1, D)
    b2r = b2.reshape(1, D)

    sc = _make_sc()
    p1lo, p1hi, s1 = _tc1(x, W1_neigh, W1_self)
    agg1lo, agg1hi, deg = sc(p1lo, p1hi, src, dst)
    p2lo, p2hi, s2 = _tc2(s1, agg1lo, agg1hi, deg, b1r, W2_neigh, W2_self)
    agg2lo, agg2hi, _ = sc(p2lo, p2hi, src, dst)
    out = _tc3(s2, agg2lo, agg2hi, deg, b2r)
    return out


# speedup vs baseline: 5.9067x; 1.2979x over previous
import functools

import jax
import jax.numpy as jnp
from jax import lax
from jax.experimental import pallas as pl
from jax.experimental.pallas import tpu as pltpu
from jax.experimental.pallas import tpu_sc as plsc

N = 10000
D = 128
E = 320000

NC = 2
NS = 16
DHALF = D // NC
EPT = E // NS
CH = 80
NCHUNK = EPT // CH
NPAD = 10240
RPT = NPAD // NS
DEGW = 16

_f32 = jnp.float32


def _sc_body(plo_hbm, phi_hbm, src_hbm, dst_hbm, agglo_out, agghi_out, deg_out,
             sidx, didx, rows0, rows1, zbuf, ones_v, zb16, agg_sh, deg_sh,
             sem0, sem1):
    c = lax.axis_index("c")
    s = lax.axis_index("s")
    rows = (rows0, rows1)
    sems = (sem0, sem1)

    def start_gather(j, b):
        @pl.when(c == 0)
        def _():
            pltpu.async_copy(plo_hbm.at[sidx.at[j]], rows[b], sems[b])

        @pl.when(c == 1)
        def _():
            pltpu.async_copy(phi_hbm.at[sidx.at[j]], rows[b], sems[b])

    def wait_gather(j, b):
        pltpu.make_async_copy(plo_hbm.at[sidx.at[j]], rows[b], sems[b]).wait()

    zero16 = jnp.zeros((16,), _f32)

    def zrow(i, carry):
        for k in range(DHALF // 16):
            zbuf[i, pl.ds(16 * k, 16)] = zero16
        return carry

    lax.fori_loop(0, 128, zrow, 0)
    for k in range(RPT // 128):
        pltpu.sync_copy(zbuf, agg_sh.at[pl.ds(s * RPT + k * 128, 128)])

    @pl.when(c == 0)
    def _():
        one16 = jnp.ones((16,), _f32)

        def zrow16(i, carry):
            zb16[i, :] = zero16
            return carry

        lax.fori_loop(0, 128, zrow16, 0)
        for k in range(RPT // 128):
            pltpu.sync_copy(zb16, deg_sh.at[pl.ds(s * RPT + k * 128, 128)])

        def orow(i, carry):
            ones_v[i, :] = one16
            return carry

        lax.fori_loop(0, CH, orow, 0)

    pltpu.sync_copy(src_hbm.at[s], sidx)
    pltpu.sync_copy(dst_hbm.at[s], didx)
    start_gather(0, 0)

    plsc.subcore_barrier()

    def body(j2, carry):
        for b in range(2):
            j = 2 * j2 + b
            wait_gather(j, b)

            @pl.when(j + 1 < NCHUNK)
            def _():
                start_gather(j + 1, 1 - b)

            pltpu.sync_copy(rows[b], agg_sh.at[didx.at[j]], add=True)

            @pl.when(c == 0)
            def _():
                pltpu.sync_copy(ones_v, deg_sh.at[didx.at[j]], add=True)

        return carry

    lax.fori_loop(0, NCHUNK // 2, body, 0)

    plsc.subcore_barrier()

    my_rows = agg_sh.at[pl.ds(s * RPT, RPT)]

    @pl.when(c == 0)
    def _():
        pltpu.sync_copy(my_rows, agglo_out.at[pl.ds(s * RPT, RPT)])
        pltpu.sync_copy(deg_sh.at[pl.ds(s * RPT, RPT)],
                        deg_out.at[pl.ds(s * RPT, RPT)])

    @pl.when(c == 1)
    def _():
        pltpu.sync_copy(my_rows, agghi_out.at[pl.ds(s * RPT, RPT)])


@functools.lru_cache(maxsize=None)
def _make_sc():
    out_type = [
        jax.ShapeDtypeStruct((NPAD, DHALF), _f32),
        jax.ShapeDtypeStruct((NPAD, DHALF), _f32),
        jax.ShapeDtypeStruct((NPAD, DEGW), _f32),
    ]
    scratch = [
        pltpu.VMEM((NCHUNK, CH), jnp.int32),
        pltpu.VMEM((NCHUNK, CH), jnp.int32),
        pltpu.VMEM((CH, DHALF), _f32),
        pltpu.VMEM((CH, DHALF), _f32),
        pltpu.VMEM((128, DHALF), _f32),
        pltpu.VMEM((CH, DEGW), _f32),
        pltpu.VMEM((128, DEGW), _f32),
        pltpu.VMEM_SHARED((NPAD, DHALF), _f32),
        pltpu.VMEM_SHARED((NPAD, DEGW), _f32),
        pltpu.SemaphoreType.DMA,
        pltpu.SemaphoreType.DMA,
    ]
    mesh = plsc.VectorSubcoreMesh(core_axis_name="c", subcore_axis_name="s")
    return pl.kernel(
        _sc_body,
        out_type=out_type,
        mesh=mesh,
        scratch_types=scratch,
        compiler_params=pltpu.CompilerParams(use_tc_tiling_on_sc=False),
        name="sage_sc_agg",
    )


_BM = 256
_GRID = (N + _BM - 1) // _BM

_dot = functools.partial(jnp.dot, preferred_element_type=_f32,
                         precision=jax.lax.Precision.HIGHEST)


def _tc1_body(x_ref, wn_ref, ws_ref, plo_ref, phi_ref, s_ref):
    xb = x_ref[...]
    p = _dot(xb, wn_ref[...])
    plo_ref[...] = p[:, :DHALF]
    phi_ref[...] = p[:, DHALF:]
    s_ref[...] = _dot(xb, ws_ref[...])


def _tc2_body(s1_ref, agglo_ref, agghi_ref, deg_ref, b_ref, wn_ref, ws_ref,
              plo_ref, phi_ref, s2_ref):
    deg = jnp.maximum(deg_ref[:, 0:1], 1.0)
    agg = jnp.concatenate([agglo_ref[...], agghi_ref[...]], axis=1)
    h = jnp.maximum(s1_ref[...] + agg / deg + b_ref[...], 0.0)
    p = _dot(h, wn_ref[...])
    plo_ref[...] = p[:, :DHALF]
    phi_ref[...] = p[:, DHALF:]
    s2_ref[...] = _dot(h, ws_ref[...])


def _tc3_body(s2_ref, agglo_ref, agghi_ref, deg_ref, b_ref, o_ref):
    deg = jnp.maximum(deg_ref[:, 0:1], 1.0)
    agg = jnp.concatenate([agglo_ref[...], agghi_ref[...]], axis=1)
    o_ref[...] = s2_ref[...] + agg / deg + b_ref[...]


_row_spec = pl.BlockSpec((_BM, D), lambda i: (i, 0))
_half_spec = pl.BlockSpec((_BM, DHALF), lambda i: (i, 0))
_w_spec = pl.BlockSpec((D, D), lambda i: (0, 0))
_b_spec = pl.BlockSpec((1, D), lambda i: (0, 0))
_deg_spec = pl.BlockSpec((_BM, DEGW), lambda i: (i, 0))
_row_out = jax.ShapeDtypeStruct((N, D), _f32)
_half_out = jax.ShapeDtypeStruct((N, DHALF), _f32)

_tc1 = pl.pallas_call(
    _tc1_body,
    grid=(_GRID,),
    in_specs=[_row_spec, _w_spec, _w_spec],
    out_specs=[_half_spec, _half_spec, _row_spec],
    out_shape=[_half_out, _half_out, _row_out],
)

_tc2 = pl.pallas_call(
    _tc2_body,
    grid=(_GRID,),
    in_specs=[_row_spec, _half_spec, _half_spec, _deg_spec, _b_spec,
              _w_spec, _w_spec],
    out_specs=[_half_spec, _half_spec, _row_spec],
    out_shape=[_half_out, _half_out, _row_out],
)

_tc3 = pl.pallas_call(
    _tc3_body,
    grid=(_GRID,),
    in_specs=[_row_spec, _half_spec, _half_spec, _deg_spec, _b_spec],
    out_specs=_row_spec,
    out_shape=_row_out,
)


def kernel(x, edge_index, W1_self, W1_neigh, b1, W2_self, W2_neigh, b2):
    src = edge_index[0].reshape(NS, NCHUNK, CH)
    dst = edge_index[1].reshape(NS, NCHUNK, CH)
    b1r = b1.reshape(1, D)
    b2r = b2.reshape(1, D)

    sc = _make_sc()
    p1lo, p1hi, s1 = _tc1(x, W1_neigh, W1_self)
    agg1lo, agg1hi, deg = sc(p1lo, p1hi, src, dst)
    p2lo, p2hi, s2 = _tc2(s1, agg1lo, agg1hi, deg, b1r, W2_neigh, W2_self)
    agg2lo, agg2hi, _ = sc(p2lo, p2hi, src, dst)
    out = _tc3(s2, agg2lo, agg2hi, deg, b2r)
    return out


# speedup vs baseline: 8.2517x; 1.3970x over previous
import functools

import jax
import jax.numpy as jnp
from jax import lax
from jax.experimental import pallas as pl
from jax.experimental.pallas import tpu as pltpu
from jax.experimental.pallas import tpu_sc as plsc

N = 10000
D = 128
E = 320000

NC = 2
NS = 16
DHALF = D // NC
EPT = E // NS
CH = 80
NCHUNK = EPT // CH
NBUF = 4
NQUAD = NCHUNK // NBUF
NPAD = 10240
RPT = NPAD // NS
DEGW = 16

_f32 = jnp.float32


def _sc_body(with_deg, *refs):
    if with_deg:
        (plo_hbm, phi_hbm, src_hbm, dst_hbm, agglo_out, agghi_out, deg_out,
         sidx, didx, r0, r1, r2, r3, zbuf, ones_v, zb16, agg_sh, deg_sh,
         g0, g1, g2, g3, s0, s1, s2, s3, dsem) = refs
    else:
        (plo_hbm, phi_hbm, src_hbm, dst_hbm, agglo_out, agghi_out,
         sidx, didx, r0, r1, r2, r3, zbuf, agg_sh,
         g0, g1, g2, g3, s0, s1, s2, s3) = refs
    c = lax.axis_index("c")
    s = lax.axis_index("s")
    rows = (r0, r1, r2, r3)
    gsem = (g0, g1, g2, g3)
    ssem = (s0, s1, s2, s3)

    def start_gather(j, b):
        @pl.when(c == 0)
        def _():
            pltpu.async_copy(plo_hbm.at[sidx.at[j]], rows[b], gsem[b])

        @pl.when(c == 1)
        def _():
            pltpu.async_copy(phi_hbm.at[sidx.at[j]], rows[b], gsem[b])

    def wait_gather(j, b):
        pltpu.make_async_copy(plo_hbm.at[sidx.at[j]], rows[b], gsem[b]).wait()

    def wait_scatter(j, b):
        pltpu.make_async_copy(
            rows[b], agg_sh.at[didx.at[j]], ssem[b]).wait()

    def step(j, b):
        wait_gather(j, b)
        pltpu.async_copy(rows[b], agg_sh.at[didx.at[j]], ssem[b], add=True)
        if with_deg:
            @pl.when(c == 0)
            def _():
                pltpu.async_copy(ones_v, deg_sh.at[didx.at[j]], dsem,
                                 add=True)
        nb = (b + 2) % NBUF

        @pl.when(j >= 2)
        def _():
            wait_scatter(j - 2, nb)

        start_gather(j + 2, nb)

    zero16 = jnp.zeros((16,), _f32)

    def zrow(i, carry):
        for k in range(DHALF // 16):
            zbuf[i, pl.ds(16 * k, 16)] = zero16
        return carry

    pltpu.sync_copy(src_hbm.at[s], sidx)
    pltpu.sync_copy(dst_hbm.at[s], didx)
    start_gather(0, 0)
    start_gather(1, 1)

    lax.fori_loop(0, 128, zrow, 0)
    for k in range(RPT // 128):
        pltpu.sync_copy(zbuf, agg_sh.at[pl.ds(s * RPT + k * 128, 128)])

    if with_deg:
        @pl.when(c == 0)
        def _():
            one16 = jnp.ones((16,), _f32)

            def zrow16(i, carry):
                zb16[i, :] = zero16
                return carry

            lax.fori_loop(0, 128, zrow16, 0)
            for k in range(RPT // 128):
                pltpu.sync_copy(zb16, deg_sh.at[pl.ds(s * RPT + k * 128, 128)])

            def orow(i, carry):
                ones_v[i, :] = one16
                return carry

            lax.fori_loop(0, CH, orow, 0)

    plsc.subcore_barrier()

    def body(q, carry):
        for b in range(NBUF):
            step(NBUF * q + b, b)
        return carry

    lax.fori_loop(0, NQUAD, body, 0)
    for j in range(NBUF * NQUAD, NCHUNK):
        b = j % NBUF
        wait_gather(j, b)
        pltpu.async_copy(rows[b], agg_sh.at[didx.at[j]], ssem[b], add=True)
        if with_deg:
            @pl.when(c == 0)
            def _():
                pltpu.async_copy(ones_v, deg_sh.at[didx.at[j]], dsem,
                                 add=True)
    for j in range(NCHUNK - NBUF, NCHUNK):
        wait_scatter(j, j % NBUF)
    if with_deg:
        @pl.when(c == 0)
        def _():
            def drain(j, carry):
                pltpu.make_async_copy(
                    ones_v, deg_sh.at[didx.at[0]], dsem).wait()
                return carry

            lax.fori_loop(0, NCHUNK, drain, 0)

    plsc.subcore_barrier()

    my_rows = agg_sh.at[pl.ds(s * RPT, RPT)]

    @pl.when(c == 0)
    def _():
        pltpu.sync_copy(my_rows, agglo_out.at[pl.ds(s * RPT, RPT)])
        if with_deg:
            pltpu.sync_copy(deg_sh.at[pl.ds(s * RPT, RPT)],
                            deg_out.at[pl.ds(s * RPT, RPT)])

    @pl.when(c == 1)
    def _():
        pltpu.sync_copy(my_rows, agghi_out.at[pl.ds(s * RPT, RPT)])


@functools.lru_cache(maxsize=None)
def _make_sc(with_deg):
    out_type = [
        jax.ShapeDtypeStruct((NPAD, DHALF), _f32),
        jax.ShapeDtypeStruct((NPAD, DHALF), _f32),
    ]
    if with_deg:
        out_type.append(jax.ShapeDtypeStruct((NPAD, DEGW), _f32))
    scratch = [
        pltpu.VMEM((NCHUNK, CH), jnp.int32),
        pltpu.VMEM((NCHUNK, CH), jnp.int32),
        pltpu.VMEM((CH, DHALF), _f32),
        pltpu.VMEM((CH, DHALF), _f32),
        pltpu.VMEM((CH, DHALF), _f32),
        pltpu.VMEM((CH, DHALF), _f32),
        pltpu.VMEM((128, DHALF), _f32),
    ]
    if with_deg:
        scratch += [
            pltpu.VMEM((CH, DEGW), _f32),
            pltpu.VMEM((128, DEGW), _f32),
        ]
    scratch.append(pltpu.VMEM_SHARED((NPAD, DHALF), _f32))
    if with_deg:
        scratch.append(pltpu.VMEM_SHARED((NPAD, DEGW), _f32))
    scratch += [pltpu.SemaphoreType.DMA] * (9 if with_deg else 8)
    mesh = plsc.VectorSubcoreMesh(core_axis_name="c", subcore_axis_name="s")
    return pl.kernel(
        functools.partial(_sc_body, with_deg),
        out_type=out_type,
        mesh=mesh,
        scratch_types=scratch,
        compiler_params=pltpu.CompilerParams(use_tc_tiling_on_sc=False),
        name="sage_sc_agg_deg" if with_deg else "sage_sc_agg",
    )


_BM = 256
_GRID = (N + _BM - 1) // _BM

_dot = functools.partial(jnp.dot, preferred_element_type=_f32,
                         precision=jax.lax.Precision.HIGHEST)


def _tc1_body(x_ref, wn_ref, ws_ref, plo_ref, phi_ref, s_ref):
    xb = x_ref[...]
    p = _dot(xb, wn_ref[...])
    plo_ref[...] = p[:, :DHALF]
    phi_ref[...] = p[:, DHALF:]
    s_ref[...] = _dot(xb, ws_ref[...])


def _tc2_body(s1_ref, agglo_ref, agghi_ref, deg_ref, b_ref, wn_ref, ws_ref,
              plo_ref, phi_ref, s2_ref):
    deg = jnp.maximum(deg_ref[:, 0:1], 1.0)
    agg = jnp.concatenate([agglo_ref[...], agghi_ref[...]], axis=1)
    h = jnp.maximum(s1_ref[...] + agg / deg + b_ref[...], 0.0)
    p = _dot(h, wn_ref[...])
    plo_ref[...] = p[:, :DHALF]
    phi_ref[...] = p[:, DHALF:]
    s2_ref[...] = _dot(h, ws_ref[...])


def _tc3_body(s2_ref, agglo_ref, agghi_ref, deg_ref, b_ref, o_ref):
    deg = jnp.maximum(deg_ref[:, 0:1], 1.0)
    agg = jnp.concatenate([agglo_ref[...], agghi_ref[...]], axis=1)
    o_ref[...] = s2_ref[...] + agg / deg + b_ref[...]


_row_spec = pl.BlockSpec((_BM, D), lambda i: (i, 0))
_half_spec = pl.BlockSpec((_BM, DHALF), lambda i: (i, 0))
_w_spec = pl.BlockSpec((D, D), lambda i: (0, 0))
_b_spec = pl.BlockSpec((1, D), lambda i: (0, 0))
_deg_spec = pl.BlockSpec((_BM, DEGW), lambda i: (i, 0))
_row_out = jax.ShapeDtypeStruct((N, D), _f32)
_half_out = jax.ShapeDtypeStruct((N, DHALF), _f32)

_tc1 = pl.pallas_call(
    _tc1_body,
    grid=(_GRID,),
    in_specs=[_row_spec, _w_spec, _w_spec],
    out_specs=[_half_spec, _half_spec, _row_spec],
    out_shape=[_half_out, _half_out, _row_out],
)

_tc2 = pl.pallas_call(
    _tc2_body,
    grid=(_GRID,),
    in_specs=[_row_spec, _half_spec, _half_spec, _deg_spec, _b_spec,
              _w_spec, _w_spec],
    out_specs=[_half_spec, _half_spec, _row_spec],
    out_shape=[_half_out, _half_out, _row_out],
)

_tc3 = pl.pallas_call(
    _tc3_body,
    grid=(_GRID,),
    in_specs=[_row_spec, _half_spec, _half_spec, _deg_spec, _b_spec],
    out_specs=_row_spec,
    out_shape=_row_out,
)


def kernel(x, edge_index, W1_self, W1_neigh, b1, W2_self, W2_neigh, b2):
    src = edge_index[0].reshape(NS, NCHUNK, CH)
    dst = edge_index[1].reshape(NS, NCHUNK, CH)
    b1r = b1.reshape(1, D)
    b2r = b2.reshape(1, D)

    p1lo, p1hi, s1 = _tc1(x, W1_neigh, W1_self)
    agg1lo, agg1hi, deg = _make_sc(True)(p1lo, p1hi, src, dst)
    p2lo, p2hi, s2 = _tc2(s1, agg1lo, agg1hi, deg, b1r, W2_neigh, W2_self)
    agg2lo, agg2hi = _make_sc(False)(p2lo, p2hi, src, dst)
    out = _tc3(s2, agg2lo, agg2hi, deg, b2r)
    return out


# speedup vs baseline: 10.4247x; 1.2633x over previous
import functools

import jax
import jax.numpy as jnp
from jax import lax
from jax.experimental import pallas as pl
from jax.experimental.pallas import tpu as pltpu
from jax.experimental.pallas import tpu_sc as plsc

N = 10000
D = 128
E = 320000

NC = 2
NS = 16
NW = NC * NS
EPW = E // NW
CH = 80
NCHUNK = EPW // CH
NBUF = 4
NQUAD = NCHUNK // NBUF
NPAD = 10240
RPT = NPAD // NS
DEGW = 16

_f32 = jnp.float32
_bf16 = jnp.bfloat16


def _sc_body(with_deg, *refs):
    if with_deg:
        (p_hbm, src_hbm, dst_hbm, agg_out, deg_out,
         sidx, didx, r0, r1, r2, r3, zbuf, ones_v, zb16, agg_sh, deg_sh,
         g0, g1, g2, g3, s0, s1, s2, s3, dsem) = refs
    else:
        (p_hbm, src_hbm, dst_hbm, agg_out,
         sidx, didx, r0, r1, r2, r3, zbuf, agg_sh,
         g0, g1, g2, g3, s0, s1, s2, s3) = refs
    c = lax.axis_index("c")
    s = lax.axis_index("s")
    wid = c * NS + s
    rows = (r0, r1, r2, r3)
    gsem = (g0, g1, g2, g3)
    ssem = (s0, s1, s2, s3)

    def start_gather(j, b):
        pltpu.async_copy(p_hbm.at[sidx.at[j]], rows[b], gsem[b])

    def wait_gather(j, b):
        pltpu.make_async_copy(p_hbm.at[sidx.at[j]], rows[b], gsem[b]).wait()

    def wait_scatter(j, b):
        pltpu.make_async_copy(rows[b], agg_sh.at[didx.at[j]], ssem[b]).wait()

    def step(j, b):
        wait_gather(j, b)
        pltpu.async_copy(rows[b], agg_sh.at[didx.at[j]], ssem[b], add=True)
        if with_deg:
            pltpu.async_copy(ones_v, deg_sh.at[didx.at[j]], dsem, add=True)
        nb = (b + 2) % NBUF

        @pl.when(j >= 2)
        def _():
            wait_scatter(j - 2, nb)

        @pl.when(j + 2 < NCHUNK)
        def _():
            start_gather(j + 2, nb)

    zero32 = jnp.zeros((32,), _bf16)

    def zrow(i, carry):
        for k in range(D // 32):
            zbuf[i, pl.ds(32 * k, 32)] = zero32
        return carry

    pltpu.sync_copy(src_hbm.at[wid], sidx)
    pltpu.sync_copy(dst_hbm.at[wid], didx)
    start_gather(0, 0)
    start_gather(1, 1)

    lax.fori_loop(0, 128, zrow, 0)
    for k in range(RPT // 128):
        pltpu.sync_copy(zbuf, agg_sh.at[pl.ds(s * RPT + k * 128, 128)])

    if with_deg:
        zero16 = jnp.zeros((16,), _f32)
        one16 = jnp.ones((16,), _f32)

        def zrow16(i, carry):
            zb16[i, :] = zero16
            return carry

        lax.fori_loop(0, 128, zrow16, 0)
        for k in range(RPT // 128):
            pltpu.sync_copy(zb16, deg_sh.at[pl.ds(s * RPT + k * 128, 128)])

        def orow(i, carry):
            ones_v[i, :] = one16
            return carry

        lax.fori_loop(0, CH, orow, 0)

    plsc.subcore_barrier()

    def body(q, carry):
        for b in range(NBUF):
            step(NBUF * q + b, b)
        return carry

    lax.fori_loop(0, NQUAD, body, 0)
    for j in range(NBUF * NQUAD, NCHUNK):
        b = j % NBUF
        wait_gather(j, b)
        pltpu.async_copy(rows[b], agg_sh.at[didx.at[j]], ssem[b], add=True)
        if with_deg:
            pltpu.async_copy(ones_v, deg_sh.at[didx.at[j]], dsem, add=True)
    for j in range(max(NBUF * NQUAD - 2, 0), NCHUNK):
        wait_scatter(j, j % NBUF)
    if with_deg:
        def drain(j, carry):
            pltpu.make_async_copy(ones_v, deg_sh.at[didx.at[0]], dsem).wait()
            return carry

        lax.fori_loop(0, NCHUNK, drain, 0)

    plsc.subcore_barrier()

    pltpu.sync_copy(agg_sh.at[pl.ds(s * RPT, RPT)],
                    agg_out.at[c, pl.ds(s * RPT, RPT)])
    if with_deg:
        pltpu.sync_copy(deg_sh.at[pl.ds(s * RPT, RPT)],
                        deg_out.at[c, pl.ds(s * RPT, RPT)])


@functools.lru_cache(maxsize=None)
def _make_sc(with_deg):
    out_type = [jax.ShapeDtypeStruct((NC, NPAD, D), _bf16)]
    if with_deg:
        out_type.append(jax.ShapeDtypeStruct((NC, NPAD, DEGW), _f32))
    scratch = [
        pltpu.VMEM((NCHUNK, CH), jnp.int32),
        pltpu.VMEM((NCHUNK, CH), jnp.int32),
        pltpu.VMEM((CH, D), _bf16),
        pltpu.VMEM((CH, D), _bf16),
        pltpu.VMEM((CH, D), _bf16),
        pltpu.VMEM((CH, D), _bf16),
        pltpu.VMEM((128, D), _bf16),
    ]
    if with_deg:
        scratch += [
            pltpu.VMEM((CH, DEGW), _f32),
            pltpu.VMEM((128, DEGW), _f32),
        ]
    scratch.append(pltpu.VMEM_SHARED((NPAD, D), _bf16))
    if with_deg:
        scratch.append(pltpu.VMEM_SHARED((NPAD, DEGW), _f32))
    scratch += [pltpu.SemaphoreType.DMA] * (9 if with_deg else 8)
    mesh = plsc.VectorSubcoreMesh(core_axis_name="c", subcore_axis_name="s")
    return pl.kernel(
        functools.partial(_sc_body, with_deg),
        out_type=out_type,
        mesh=mesh,
        scratch_types=scratch,
        compiler_params=pltpu.CompilerParams(use_tc_tiling_on_sc=False),
        name="sage_sc_agg_deg" if with_deg else "sage_sc_agg",
    )


_BM = 256
_GRID = (N + _BM - 1) // _BM

_dot = functools.partial(jnp.dot, preferred_element_type=_f32,
                         precision=jax.lax.Precision.HIGHEST)


def _tc1_body(x_ref, wn_ref, ws_ref, p_ref, s_ref):
    xb = x_ref[...]
    p_ref[...] = _dot(xb, wn_ref[...]).astype(_bf16)
    s_ref[...] = _dot(xb, ws_ref[...])


def _tc2_body(s1_ref, agg_ref, deg_ref, b_ref, wn_ref, ws_ref,
              p_ref, s2_ref):
    deg = jnp.maximum(deg_ref[0, :, 0:1] + deg_ref[1, :, 0:1], 1.0)
    agg = agg_ref[0].astype(_f32) + agg_ref[1].astype(_f32)
    h = jnp.maximum(s1_ref[...] + agg / deg + b_ref[...], 0.0)
    p_ref[...] = _dot(h, wn_ref[...]).astype(_bf16)
    s2_ref[...] = _dot(h, ws_ref[...])


def _tc3_body(s2_ref, agg_ref, deg_ref, b_ref, o_ref):
    deg = jnp.maximum(deg_ref[0, :, 0:1] + deg_ref[1, :, 0:1], 1.0)
    agg = agg_ref[0].astype(_f32) + agg_ref[1].astype(_f32)
    o_ref[...] = s2_ref[...] + agg / deg + b_ref[...]


_row_spec = pl.BlockSpec((_BM, D), lambda i: (i, 0))
_w_spec = pl.BlockSpec((D, D), lambda i: (0, 0))
_b_spec = pl.BlockSpec((1, D), lambda i: (0, 0))
_agg_spec = pl.BlockSpec((NC, _BM, D), lambda i: (0, i, 0))
_deg_spec = pl.BlockSpec((NC, _BM, DEGW), lambda i: (0, i, 0))
_row_out = jax.ShapeDtypeStruct((N, D), _f32)
_p_out = jax.ShapeDtypeStruct((N, D), _bf16)

_tc1 = pl.pallas_call(
    _tc1_body,
    grid=(_GRID,),
    in_specs=[_row_spec, _w_spec, _w_spec],
    out_specs=[_row_spec, _row_spec],
    out_shape=[_p_out, _row_out],
)

_tc2 = pl.pallas_call(
    _tc2_body,
    grid=(_GRID,),
    in_specs=[_row_spec, _agg_spec, _deg_spec, _b_spec, _w_spec, _w_spec],
    out_specs=[_row_spec, _row_spec],
    out_shape=[_p_out, _row_out],
)

_tc3 = pl.pallas_call(
    _tc3_body,
    grid=(_GRID,),
    in_specs=[_row_spec, _agg_spec, _deg_spec, _b_spec],
    out_specs=_row_spec,
    out_shape=_row_out,
)


def kernel(x, edge_index, W1_self, W1_neigh, b1, W2_self, W2_neigh, b2):
    src = edge_index[0].reshape(NW, NCHUNK, CH)
    dst = edge_index[1].reshape(NW, NCHUNK, CH)
    b1r = b1.reshape(1, D)
    b2r = b2.reshape(1, D)

    p1, s1 = _tc1(x, W1_neigh, W1_self)
    agg1, deg = _make_sc(True)(p1, src, dst)
    p2, s2 = _tc2(s1, agg1, deg, b1r, W2_neigh, W2_self)
    (agg2,) = _make_sc(False)(p2, src, dst)
    out = _tc3(s2, agg2, deg, b2r)
    return out


# speedup vs baseline: 11.6018x; 1.1129x over previous
import functools

import jax
import jax.numpy as jnp
from jax import lax
from jax.experimental import pallas as pl
from jax.experimental.pallas import tpu as pltpu
from jax.experimental.pallas import tpu_sc as plsc

N = 10000
D = 128
E = 320000

NC = 2
NS = 16
NW = NC * NS
EPW = E // NW
CH = 80
NCHUNK = EPW // CH
NBUF = 5
LA = NBUF - 2
NQUAD = NCHUNK // NBUF
NPAD = 10240
RPT = NPAD // NS
DEGW = 16

_f32 = jnp.float32
_bf16 = jnp.bfloat16


def _sc_body(with_deg, *refs):
    if with_deg:
        (p_hbm, src_hbm, dst_hbm, agg_out, deg_out,
         sidx, didx, r0, r1, r2, r3, r4, zbuf, ones_v, zb16, agg_sh, deg_sh,
         g0, g1, g2, g3, g4, s0, s1, s2, s3, s4, dsem) = refs
    else:
        (p_hbm, src_hbm, dst_hbm, agg_out,
         sidx, didx, r0, r1, r2, r3, r4, zbuf, agg_sh,
         g0, g1, g2, g3, g4, s0, s1, s2, s3, s4) = refs
    c = lax.axis_index("c")
    s = lax.axis_index("s")
    wid = c * NS + s
    rows = (r0, r1, r2, r3, r4)
    gsem = (g0, g1, g2, g3, g4)
    ssem = (s0, s1, s2, s3, s4)

    def start_gather(j, b):
        pltpu.async_copy(p_hbm.at[sidx.at[j]], rows[b], gsem[b])

    def wait_gather(j, b):
        pltpu.make_async_copy(p_hbm.at[sidx.at[j]], rows[b], gsem[b]).wait()

    def wait_scatter(j, b):
        pltpu.make_async_copy(rows[b], agg_sh.at[didx.at[j]], ssem[b]).wait()

    def step(j, b):
        wait_gather(j, b)
        pltpu.async_copy(rows[b], agg_sh.at[didx.at[j]], ssem[b], add=True)
        if with_deg:
            pltpu.async_copy(ones_v, deg_sh.at[didx.at[j]], dsem, add=True)
        nb = (b + LA) % NBUF

        @pl.when(j >= 2)
        def _():
            wait_scatter(j - 2, nb)

        @pl.when(j + LA < NCHUNK)
        def _():
            start_gather(j + LA, nb)

    zero32 = jnp.zeros((32,), _bf16)

    def zrow(i, carry):
        for k in range(D // 32):
            zbuf[i, pl.ds(32 * k, 32)] = zero32
        return carry

    pltpu.sync_copy(src_hbm.at[wid], sidx)
    pltpu.sync_copy(dst_hbm.at[wid], didx)
    for j in range(LA):
        start_gather(j, j)

    lax.fori_loop(0, 128, zrow, 0)
    for k in range(RPT // 128):
        pltpu.sync_copy(zbuf, agg_sh.at[pl.ds(s * RPT + k * 128, 128)])

    if with_deg:
        zero16 = jnp.zeros((16,), _f32)
        one16 = jnp.ones((16,), _f32)

        def zrow16(i, carry):
            zb16[i, :] = zero16
            return carry

        lax.fori_loop(0, 128, zrow16, 0)
        for k in range(RPT // 128):
            pltpu.sync_copy(zb16, deg_sh.at[pl.ds(s * RPT + k * 128, 128)])

        def orow(i, carry):
            ones_v[i, :] = one16
            return carry

        lax.fori_loop(0, CH, orow, 0)

    plsc.subcore_barrier()

    def body(q, carry):
        for b in range(NBUF):
            step(NBUF * q + b, b)
        return carry

    lax.fori_loop(0, NQUAD, body, 0)
    for j in range(NCHUNK - 2, NCHUNK):
        wait_scatter(j, j % NBUF)
    if with_deg:
        def drain(j, carry):
            pltpu.make_async_copy(ones_v, deg_sh.at[didx.at[0]], dsem).wait()
            return carry

        lax.fori_loop(0, NCHUNK, drain, 0)

    plsc.subcore_barrier()

    pltpu.sync_copy(agg_sh.at[pl.ds(s * RPT, RPT)],
                    agg_out.at[c, pl.ds(s * RPT, RPT)])
    if with_deg:
        pltpu.sync_copy(deg_sh.at[pl.ds(s * RPT, RPT)],
                        deg_out.at[c, pl.ds(s * RPT, RPT)])


@functools.lru_cache(maxsize=None)
def _make_sc(with_deg):
    out_type = [jax.ShapeDtypeStruct((NC, NPAD, D), _bf16)]
    if with_deg:
        out_type.append(jax.ShapeDtypeStruct((NC, NPAD, DEGW), _f32))
    scratch = [
        pltpu.VMEM((NCHUNK, CH), jnp.int32),
        pltpu.VMEM((NCHUNK, CH), jnp.int32),
        pltpu.VMEM((CH, D), _bf16),
        pltpu.VMEM((CH, D), _bf16),
        pltpu.VMEM((CH, D), _bf16),
        pltpu.VMEM((CH, D), _bf16),
        pltpu.VMEM((CH, D), _bf16),
        pltpu.VMEM((128, D), _bf16),
    ]
    if with_deg:
        scratch += [
            pltpu.VMEM((CH, DEGW), _f32),
            pltpu.VMEM((128, DEGW), _f32),
        ]
    scratch.append(pltpu.VMEM_SHARED((NPAD, D), _bf16))
    if with_deg:
        scratch.append(pltpu.VMEM_SHARED((NPAD, DEGW), _f32))
    scratch += [pltpu.SemaphoreType.DMA] * (11 if with_deg else 10)
    mesh = plsc.VectorSubcoreMesh(core_axis_name="c", subcore_axis_name="s")
    return pl.kernel(
        functools.partial(_sc_body, with_deg),
        out_type=out_type,
        mesh=mesh,
        scratch_types=scratch,
        compiler_params=pltpu.CompilerParams(use_tc_tiling_on_sc=False),
        name="sage_sc_agg_deg" if with_deg else "sage_sc_agg",
    )


_BM = 256
_GRID = (N + _BM - 1) // _BM

_dot = functools.partial(jnp.dot, preferred_element_type=_f32,
                         precision=jax.lax.Precision.HIGHEST)


def _proj_body(x_ref, w_ref, p_ref):
    p_ref[...] = _dot(x_ref[...], w_ref[...]).astype(p_ref.dtype)


def _tc2_body(s1_ref, agg_ref, deg_ref, b_ref, wn_ref, h_ref, p_ref):
    deg = jnp.maximum(deg_ref[0, :, 0:1] + deg_ref[1, :, 0:1], 1.0)
    agg = agg_ref[0].astype(_f32) + agg_ref[1].astype(_f32)
    h = jnp.maximum(s1_ref[...] + agg / deg + b_ref[...], 0.0)
    h_ref[...] = h
    p_ref[...] = _dot(h, wn_ref[...]).astype(_bf16)


def _tc3_body(s2_ref, agg_ref, deg_ref, b_ref, o_ref):
    deg = jnp.maximum(deg_ref[0, :, 0:1] + deg_ref[1, :, 0:1], 1.0)
    agg = agg_ref[0].astype(_f32) + agg_ref[1].astype(_f32)
    o_ref[...] = s2_ref[...] + agg / deg + b_ref[...]


_row_spec = pl.BlockSpec((_BM, D), lambda i: (i, 0))
_w_spec = pl.BlockSpec((D, D), lambda i: (0, 0))
_b_spec = pl.BlockSpec((1, D), lambda i: (0, 0))
_agg_spec = pl.BlockSpec((NC, _BM, D), lambda i: (0, i, 0))
_deg_spec = pl.BlockSpec((NC, _BM, DEGW), lambda i: (0, i, 0))
_row_out = jax.ShapeDtypeStruct((N, D), _f32)
_p_out = jax.ShapeDtypeStruct((N, D), _bf16)

_proj_bf16 = pl.pallas_call(
    _proj_body,
    grid=(_GRID,),
    in_specs=[_row_spec, _w_spec],
    out_specs=_row_spec,
    out_shape=_p_out,
)

_proj_f32 = pl.pallas_call(
    _proj_body,
    grid=(_GRID,),
    in_specs=[_row_spec, _w_spec],
    out_specs=_row_spec,
    out_shape=_row_out,
)

_tc2 = pl.pallas_call(
    _tc2_body,
    grid=(_GRID,),
    in_specs=[_row_spec, _agg_spec, _deg_spec, _b_spec, _w_spec],
    out_specs=[_row_spec, _row_spec],
    out_shape=[_row_out, _p_out],
)

_tc3 = pl.pallas_call(
    _tc3_body,
    grid=(_GRID,),
    in_specs=[_row_spec, _agg_spec, _deg_spec, _b_spec],
    out_specs=_row_spec,
    out_shape=_row_out,
)


def kernel(x, edge_index, W1_self, W1_neigh, b1, W2_self, W2_neigh, b2):
    src = edge_index[0].reshape(NW, NCHUNK, CH)
    dst = edge_index[1].reshape(NW, NCHUNK, CH)
    b1r = b1.reshape(1, D)
    b2r = b2.reshape(1, D)

    p1 = _proj_bf16(x, W1_neigh)
    agg1, deg = _make_sc(True)(p1, src, dst)
    s1 = _proj_f32(x, W1_self)
    h, p2 = _tc2(s1, agg1, deg, b1r, W2_neigh)
    agg2 = _make_sc(False)(p2, src, dst)[0]
    s2 = _proj_f32(h, W2_self)
    out = _tc3(s2, agg2, deg, b2r)
    return out


# speedup vs baseline: 13.3644x; 1.1519x over previous
import functools

import jax
import jax.numpy as jnp
from jax import lax
from jax.experimental import pallas as pl
from jax.experimental.pallas import tpu as pltpu
from jax.experimental.pallas import tpu_sc as plsc

N = 10000
D = 128
E = 320000

NC = 2
NS = 16
NW = NC * NS
EPW = E // NW
CH = 80
NCHUNK = EPW // CH
NBUF = 5
LA = NBUF - 2
NQUAD = NCHUNK // NBUF
NPAD = 10240
RPT = NPAD // NS
DEGW = 16

_f32 = jnp.float32
_bf16 = jnp.bfloat16


def _sc_body(with_deg, *refs):
    if with_deg:
        (p_hbm, ei_hbm, agg_out, deg_out,
         sidx, didx, r0, r1, r2, r3, r4, zbuf, ones_v, zb16, agg_sh, deg_sh,
         g0, g1, g2, g3, g4, s0, s1, s2, s3, s4, dsem) = refs
    else:
        (p_hbm, ei_hbm, agg_out,
         sidx, didx, r0, r1, r2, r3, r4, zbuf, agg_sh,
         g0, g1, g2, g3, g4, s0, s1, s2, s3, s4) = refs
    c = lax.axis_index("c")
    s = lax.axis_index("s")
    wid = c * NS + s
    rows = (r0, r1, r2, r3, r4)
    gsem = (g0, g1, g2, g3, g4)
    ssem = (s0, s1, s2, s3, s4)

    def start_gather(j, b):
        pltpu.async_copy(p_hbm.at[sidx.at[j]], rows[b], gsem[b])

    def wait_gather(j, b):
        pltpu.make_async_copy(p_hbm.at[sidx.at[j]], rows[b], gsem[b]).wait()

    def wait_scatter(j, b):
        pltpu.make_async_copy(rows[b], agg_sh.at[didx.at[j]], ssem[b]).wait()

    def step(j, b):
        wait_gather(j, b)
        pltpu.async_copy(rows[b], agg_sh.at[didx.at[j]], ssem[b], add=True)
        if with_deg:
            pltpu.async_copy(ones_v, deg_sh.at[didx.at[j]], dsem, add=True)
        nb = (b + LA) % NBUF

        @pl.when(j >= 2)
        def _():
            wait_scatter(j - 2, nb)

        @pl.when(j + LA < NCHUNK)
        def _():
            start_gather(j + LA, nb)

    zero32 = jnp.zeros((32,), _bf16)

    def zrow(i, carry):
        for k in range(D // 32):
            zbuf[i, pl.ds(32 * k, 32)] = zero32
        return carry

    pltpu.sync_copy(ei_hbm.at[0, wid], sidx)
    pltpu.sync_copy(ei_hbm.at[1, wid], didx)
    for j in range(LA):
        start_gather(j, j)

    lax.fori_loop(0, 128, zrow, 0)
    for k in range(RPT // 128):
        pltpu.sync_copy(zbuf, agg_sh.at[pl.ds(s * RPT + k * 128, 128)])

    if with_deg:
        zero16 = jnp.zeros((16,), _f32)
        one16 = jnp.ones((16,), _f32)

        def zrow16(i, carry):
            zb16[i, :] = zero16
            return carry

        lax.fori_loop(0, 128, zrow16, 0)
        for k in range(RPT // 128):
            pltpu.sync_copy(zb16, deg_sh.at[pl.ds(s * RPT + k * 128, 128)])

        def orow(i, carry):
            ones_v[i, :] = one16
            return carry

        lax.fori_loop(0, CH, orow, 0)

    plsc.subcore_barrier()

    def body(q, carry):
        for b in range(NBUF):
            step(NBUF * q + b, b)
        return carry

    lax.fori_loop(0, NQUAD, body, 0)
    for j in range(NCHUNK - 2, NCHUNK):
        wait_scatter(j, j % NBUF)
    if with_deg:
        def drain(j, carry):
            pltpu.make_async_copy(ones_v, deg_sh.at[didx.at[0]], dsem).wait()
            return carry

        lax.fori_loop(0, NCHUNK, drain, 0)

    plsc.subcore_barrier()

    pltpu.sync_copy(agg_sh.at[pl.ds(s * RPT, RPT)],
                    agg_out.at[c, pl.ds(s * RPT, RPT)])
    if with_deg:
        pltpu.sync_copy(deg_sh.at[pl.ds(s * RPT, RPT)],
                        deg_out.at[c, pl.ds(s * RPT, RPT)])


@functools.lru_cache(maxsize=None)
def _make_sc(with_deg):
    out_type = [jax.ShapeDtypeStruct((NC, NPAD, D), _bf16)]
    if with_deg:
        out_type.append(jax.ShapeDtypeStruct((NC, NPAD, DEGW), _f32))
    scratch = [
        pltpu.VMEM((NCHUNK, CH), jnp.int32),
        pltpu.VMEM((NCHUNK, CH), jnp.int32),
        pltpu.VMEM((CH, D), _bf16),
        pltpu.VMEM((CH, D), _bf16),
        pltpu.VMEM((CH, D), _bf16),
        pltpu.VMEM((CH, D), _bf16),
        pltpu.VMEM((CH, D), _bf16),
        pltpu.VMEM((128, D), _bf16),
    ]
    if with_deg:
        scratch += [
            pltpu.VMEM((CH, DEGW), _f32),
            pltpu.VMEM((128, DEGW), _f32),
        ]
    scratch.append(pltpu.VMEM_SHARED((NPAD, D), _bf16))
    if with_deg:
        scratch.append(pltpu.VMEM_SHARED((NPAD, DEGW), _f32))
    scratch += [pltpu.SemaphoreType.DMA] * (11 if with_deg else 10)
    mesh = plsc.VectorSubcoreMesh(core_axis_name="c", subcore_axis_name="s")
    return pl.kernel(
        functools.partial(_sc_body, with_deg),
        out_type=out_type,
        mesh=mesh,
        scratch_types=scratch,
        compiler_params=pltpu.CompilerParams(use_tc_tiling_on_sc=False),
        name="sage_sc_agg_deg" if with_deg else "sage_sc_agg",
    )


_BM = 512
_GRID = (N + _BM - 1) // _BM

_dot = functools.partial(jnp.dot, preferred_element_type=_f32,
                         precision=jax.lax.Precision.HIGHEST)


def _proj_body(x_ref, w_ref, p_ref):
    p_ref[...] = _dot(x_ref[...], w_ref[...]).astype(p_ref.dtype)


def _tc2_body(s1_ref, agg_ref, deg_ref, b_ref, wn_ref, h_ref, p_ref):
    deg = jnp.maximum(deg_ref[0, :, 0:1] + deg_ref[1, :, 0:1], 1.0)
    agg = agg_ref[0].astype(_f32) + agg_ref[1].astype(_f32)
    h = jnp.maximum(s1_ref[...] + agg / deg + b_ref[...], 0.0)
    h_ref[...] = h
    p_ref[...] = _dot(h, wn_ref[...]).astype(_bf16)


def _tc3_body(s2_ref, agg_ref, deg_ref, b_ref, o_ref):
    deg = jnp.maximum(deg_ref[0, :, 0:1] + deg_ref[1, :, 0:1], 1.0)
    agg = agg_ref[0].astype(_f32) + agg_ref[1].astype(_f32)
    o_ref[...] = s2_ref[...] + agg / deg + b_ref[...]


_row_spec = pl.BlockSpec((_BM, D), lambda i: (i, 0))
_w_spec = pl.BlockSpec((D, D), lambda i: (0, 0))
_b_spec = pl.BlockSpec((1, D), lambda i: (0, 0))
_agg_spec = pl.BlockSpec((NC, _BM, D), lambda i: (0, i, 0))
_deg_spec = pl.BlockSpec((NC, _BM, DEGW), lambda i: (0, i, 0))
_row_out = jax.ShapeDtypeStruct((N, D), _f32)
_p_out = jax.ShapeDtypeStruct((N, D), _bf16)

_proj_bf16 = pl.pallas_call(
    _proj_body,
    grid=(_GRID,),
    in_specs=[_row_spec, _w_spec],
    out_specs=_row_spec,
    out_shape=_p_out,
)

_proj_f32 = pl.pallas_call(
    _proj_body,
    grid=(_GRID,),
    in_specs=[_row_spec, _w_spec],
    out_specs=_row_spec,
    out_shape=_row_out,
)

_tc2 = pl.pallas_call(
    _tc2_body,
    grid=(_GRID,),
    in_specs=[_row_spec, _agg_spec, _deg_spec, _b_spec, _w_spec],
    out_specs=[_row_spec, _row_spec],
    out_shape=[_row_out, _p_out],
)

_tc3 = pl.pallas_call(
    _tc3_body,
    grid=(_GRID,),
    in_specs=[_row_spec, _agg_spec, _deg_spec, _b_spec],
    out_specs=_row_spec,
    out_shape=_row_out,
)


def kernel(x, edge_index, W1_self, W1_neigh, b1, W2_self, W2_neigh, b2):
    ei = edge_index.reshape(2, NW, NCHUNK, CH)
    b1r = b1.reshape(1, D)
    b2r = b2.reshape(1, D)

    p1 = _proj_bf16(x, W1_neigh)
    agg1, deg = _make_sc(True)(p1, ei)
    s1 = _proj_f32(x, W1_self)
    h, p2 = _tc2(s1, agg1, deg, b1r, W2_neigh)
    agg2 = _make_sc(False)(p2, ei)[0]
    s2 = _proj_f32(h, W2_self)
    out = _tc3(s2, agg2, deg, b2r)
    return out


# speedup vs baseline: 14.1985x; 1.0624x over previous
import functools

import jax
import jax.numpy as jnp
from jax import lax
from jax.experimental import pallas as pl
from jax.experimental.pallas import tpu as pltpu
from jax.experimental.pallas import tpu_sc as plsc

N = 10000
D = 128
E = 320000

NC = 2
NS = 16
NW = NC * NS
EPW = E // NW
CH = 80
NCHUNK = EPW // CH
NBUF = 5
LA = NBUF - 2
NQUAD = NCHUNK // NBUF
NPAD = 10240
RPT = NPAD // NS
DEGW = 16

_f32 = jnp.float32
_bf16 = jnp.bfloat16


def _sc_body(with_deg, *refs):
    if with_deg:
        (p_hbm, ei_hbm, agg_out, deg_out,
         sidx, didx, r0, r1, r2, r3, r4, zbuf, ones_v, zb16, agg_sh, deg_sh,
         g0, g1, g2, g3, g4, s0, s1, s2, s3, s4, dsem) = refs
    else:
        (p_hbm, ei_hbm, agg_out,
         sidx, didx, r0, r1, r2, r3, r4, zbuf, agg_sh,
         g0, g1, g2, g3, g4, s0, s1, s2, s3, s4) = refs
    c = lax.axis_index("c")
    s = lax.axis_index("s")
    wid = c * NS + s
    rows = (r0, r1, r2, r3, r4)
    gsem = (g0, g1, g2, g3, g4)
    ssem = (s0, s1, s2, s3, s4)

    def start_gather(j, b):
        pltpu.async_copy(p_hbm.at[sidx.at[j]], rows[b], gsem[b])

    def wait_gather(j, b):
        pltpu.make_async_copy(p_hbm.at[sidx.at[j]], rows[b], gsem[b]).wait()

    def wait_scatter(j, b):
        pltpu.make_async_copy(rows[b], agg_sh.at[didx.at[j]], ssem[b]).wait()

    def step(j, b):
        wait_gather(j, b)
        pltpu.async_copy(rows[b], agg_sh.at[didx.at[j]], ssem[b], add=True)
        if with_deg:
            pltpu.async_copy(ones_v, deg_sh.at[didx.at[j]], dsem, add=True)
        nb = (b + LA) % NBUF

        @pl.when(j >= 2)
        def _():
            wait_scatter(j - 2, nb)

        @pl.when(j + LA < NCHUNK)
        def _():
            start_gather(j + LA, nb)

    zero32 = jnp.zeros((32,), _bf16)

    def zrow(i, carry):
        for k in range(D // 32):
            zbuf[i, pl.ds(32 * k, 32)] = zero32
        return carry

    pltpu.sync_copy(ei_hbm.at[0, wid], sidx)
    pltpu.sync_copy(ei_hbm.at[1, wid], didx)
    for j in range(LA):
        start_gather(j, j)

    lax.fori_loop(0, 128, zrow, 0)
    for k in range(RPT // 128):
        pltpu.sync_copy(zbuf, agg_sh.at[pl.ds(s * RPT + k * 128, 128)])

    if with_deg:
        zero16 = jnp.zeros((16,), _f32)
        one16 = jnp.ones((16,), _f32)

        def zrow16(i, carry):
            zb16[i, :] = zero16
            return carry

        lax.fori_loop(0, 128, zrow16, 0)
        for k in range(RPT // 128):
            pltpu.sync_copy(zb16, deg_sh.at[pl.ds(s * RPT + k * 128, 128)])

        def orow(i, carry):
            ones_v[i, :] = one16
            return carry

        lax.fori_loop(0, CH, orow, 0)

    plsc.subcore_barrier()

    def body(q, carry):
        for b in range(NBUF):
            step(NBUF * q + b, b)
        return carry

    lax.fori_loop(0, NQUAD, body, 0)
    for j in range(NCHUNK - 2, NCHUNK):
        wait_scatter(j, j % NBUF)
    if with_deg:
        def drain(j, carry):
            pltpu.make_async_copy(ones_v, deg_sh.at[didx.at[0]], dsem).wait()
            return carry

        lax.fori_loop(0, NCHUNK, drain, 0)

    plsc.subcore_barrier()

    pltpu.sync_copy(agg_sh.at[pl.ds(s * RPT, RPT)],
                    agg_out.at[c, pl.ds(s * RPT, RPT)])
    if with_deg:
        pltpu.sync_copy(deg_sh.at[pl.ds(s * RPT, RPT)],
                        deg_out.at[c, pl.ds(s * RPT, RPT)])


@functools.lru_cache(maxsize=None)
def _make_sc(with_deg):
    out_type = [jax.ShapeDtypeStruct((NC, NPAD, D), _bf16)]
    if with_deg:
        out_type.append(jax.ShapeDtypeStruct((NC, NPAD, DEGW), _f32))
    scratch = [
        pltpu.VMEM((NCHUNK, CH), jnp.int32),
        pltpu.VMEM((NCHUNK, CH), jnp.int32),
        pltpu.VMEM((CH, D), _bf16),
        pltpu.VMEM((CH, D), _bf16),
        pltpu.VMEM((CH, D), _bf16),
        pltpu.VMEM((CH, D), _bf16),
        pltpu.VMEM((CH, D), _bf16),
        pltpu.VMEM((128, D), _bf16),
    ]
    if with_deg:
        scratch += [
            pltpu.VMEM((CH, DEGW), _f32),
            pltpu.VMEM((128, DEGW), _f32),
        ]
    scratch.append(pltpu.VMEM_SHARED((NPAD, D), _bf16))
    if with_deg:
        scratch.append(pltpu.VMEM_SHARED((NPAD, DEGW), _f32))
    scratch += [pltpu.SemaphoreType.DMA] * (11 if with_deg else 10)
    mesh = plsc.VectorSubcoreMesh(core_axis_name="c", subcore_axis_name="s")
    return pl.kernel(
        functools.partial(_sc_body, with_deg),
        out_type=out_type,
        mesh=mesh,
        scratch_types=scratch,
        compiler_params=pltpu.CompilerParams(use_tc_tiling_on_sc=False),
        name="sage_sc_agg_deg" if with_deg else "sage_sc_agg",
    )


_BM = 1024
_GRID = (N + _BM - 1) // _BM

_dot = functools.partial(jnp.dot, preferred_element_type=_f32,
                         precision=jax.lax.Precision.HIGHEST)


def _proj_body(x_ref, w_ref, p_ref):
    p_ref[...] = _dot(x_ref[...], w_ref[...]).astype(p_ref.dtype)


def _tc2_body(s1_ref, agg_ref, deg_ref, b_ref, wn_ref, h_ref, p_ref):
    deg = jnp.maximum(deg_ref[0, :, 0:1] + deg_ref[1, :, 0:1], 1.0)
    agg = agg_ref[0].astype(_f32) + agg_ref[1].astype(_f32)
    h = jnp.maximum(s1_ref[...] + agg / deg + b_ref[...], 0.0)
    h_ref[...] = h
    p_ref[...] = _dot(h, wn_ref[...]).astype(_bf16)


def _tc3_body(s2_ref, agg_ref, deg_ref, b_ref, o_ref):
    deg = jnp.maximum(deg_ref[0, :, 0:1] + deg_ref[1, :, 0:1], 1.0)
    agg = agg_ref[0].astype(_f32) + agg_ref[1].astype(_f32)
    o_ref[...] = s2_ref[...] + agg / deg + b_ref[...]


_row_spec = pl.BlockSpec((_BM, D), lambda i: (i, 0))
_w_spec = pl.BlockSpec((D, D), lambda i: (0, 0))
_b_spec = pl.BlockSpec((1, D), lambda i: (0, 0))
_agg_spec = pl.BlockSpec((NC, _BM, D), lambda i: (0, i, 0))
_deg_spec = pl.BlockSpec((NC, _BM, DEGW), lambda i: (0, i, 0))
_row_out = jax.ShapeDtypeStruct((N, D), _f32)
_p_out = jax.ShapeDtypeStruct((N, D), _bf16)

_proj_bf16 = pl.pallas_call(
    _proj_body,
    grid=(_GRID,),
    in_specs=[_row_spec, _w_spec],
    out_specs=_row_spec,
    out_shape=_p_out,
)

_proj_f32 = pl.pallas_call(
    _proj_body,
    grid=(_GRID,),
    in_specs=[_row_spec, _w_spec],
    out_specs=_row_spec,
    out_shape=_row_out,
)

_tc2 = pl.pallas_call(
    _tc2_body,
    grid=(_GRID,),
    in_specs=[_row_spec, _agg_spec, _deg_spec, _b_spec, _w_spec],
    out_specs=[_row_spec, _row_spec],
    out_shape=[_row_out, _p_out],
)

_tc3 = pl.pallas_call(
    _tc3_body,
    grid=(_GRID,),
    in_specs=[_row_spec, _agg_spec, _deg_spec, _b_spec],
    out_specs=_row_spec,
    out_shape=_row_out,
)


def kernel(x, edge_index, W1_self, W1_neigh, b1, W2_self, W2_neigh, b2):
    ei = edge_index.reshape(2, NW, NCHUNK, CH)
    b1r = b1.reshape(1, D)
    b2r = b2.reshape(1, D)

    p1 = _proj_bf16(x, W1_neigh)
    agg1, deg = _make_sc(True)(p1, ei)
    s1 = _proj_f32(x, W1_self)
    h, p2 = _tc2(s1, agg1, deg, b1r, W2_neigh)
    agg2 = _make_sc(False)(p2, ei)[0]
    s2 = _proj_f32(h, W2_self)
    out = _tc3(s2, agg2, deg, b2r)
    return out


# speedup vs baseline: 14.5067x; 1.0217x over previous
import functools

import jax
import jax.numpy as jnp
from jax import lax
from jax.experimental import pallas as pl
from jax.experimental.pallas import tpu as pltpu
from jax.experimental.pallas import tpu_sc as plsc

N = 10000
D = 128
E = 320000

NC = 2
NS = 16
NW = NC * NS
EPW = E // NW
CH = 80
NCHUNK = EPW // CH
NBUF = 5
LA = NBUF - 2
NQUAD = NCHUNK // NBUF
NPAD = 10240
RPT = NPAD // NS
DEGW = 16

_f32 = jnp.float32
_bf16 = jnp.bfloat16


def _sc_body(with_deg, *refs):
    if with_deg:
        (p_hbm, ei_hbm, agg_out, deg_out,
         sidx, didx, r0, r1, r2, r3, r4, zbuf, ones_v, zb16, agg_sh, deg_sh,
         g0, g1, g2, g3, g4, s0, s1, s2, s3, s4, dsem) = refs
    else:
        (p_hbm, ei_hbm, agg_out,
         sidx, didx, r0, r1, r2, r3, r4, zbuf, agg_sh,
         g0, g1, g2, g3, g4, s0, s1, s2, s3, s4) = refs
    c = lax.axis_index("c")
    s = lax.axis_index("s")
    wid = c * NS + s
    rows = (r0, r1, r2, r3, r4)
    gsem = (g0, g1, g2, g3, g4)
    ssem = (s0, s1, s2, s3, s4)

    def start_gather(j, b):
        pltpu.async_copy(p_hbm.at[sidx.at[j]], rows[b], gsem[b])

    def wait_gather(j, b):
        pltpu.make_async_copy(p_hbm.at[sidx.at[j]], rows[b], gsem[b]).wait()

    def wait_scatter(j, b):
        pltpu.make_async_copy(rows[b], agg_sh.at[didx.at[j]], ssem[b]).wait()

    def step(j, b):
        wait_gather(j, b)
        pltpu.async_copy(rows[b], agg_sh.at[didx.at[j]], ssem[b], add=True)
        if with_deg:
            pltpu.async_copy(ones_v, deg_sh.at[didx.at[j]], dsem, add=True)
        nb = (b + LA) % NBUF

        @pl.when(j >= 2)
        def _():
            wait_scatter(j - 2, nb)

        @pl.when(j + LA < NCHUNK)
        def _():
            start_gather(j + LA, nb)

    zero32 = jnp.zeros((32,), _bf16)

    def zrow(i, carry):
        for k in range(D // 32):
            zbuf[i, pl.ds(32 * k, 32)] = zero32
        return carry

    pltpu.sync_copy(ei_hbm.at[0, wid], sidx)
    pltpu.sync_copy(ei_hbm.at[1, wid], didx)
    for j in range(LA):
        start_gather(j, j)

    lax.fori_loop(0, 128, zrow, 0)
    for k in range(RPT // 128):
        pltpu.sync_copy(zbuf, agg_sh.at[pl.ds(s * RPT + k * 128, 128)])

    if with_deg:
        zero16 = jnp.zeros((16,), _f32)
        one16 = jnp.ones((16,), _f32)

        def zrow16(i, carry):
            zb16[i, :] = zero16
            return carry

        lax.fori_loop(0, 128, zrow16, 0)
        for k in range(RPT // 128):
            pltpu.sync_copy(zb16, deg_sh.at[pl.ds(s * RPT + k * 128, 128)])

        def orow(i, carry):
            ones_v[i, :] = one16
            return carry

        lax.fori_loop(0, CH, orow, 0)

    plsc.subcore_barrier()

    def body(q, carry):
        for b in range(NBUF):
            step(NBUF * q + b, b)
        return carry

    lax.fori_loop(0, NQUAD, body, 0)
    for j in range(NCHUNK - 2, NCHUNK):
        wait_scatter(j, j % NBUF)
    if with_deg:
        def drain(j, carry):
            pltpu.make_async_copy(ones_v, deg_sh.at[didx.at[0]], dsem).wait()
            return carry

        lax.fori_loop(0, NCHUNK, drain, 0)

    plsc.subcore_barrier()

    pltpu.sync_copy(agg_sh.at[pl.ds(s * RPT, RPT)],
                    agg_out.at[c, pl.ds(s * RPT, RPT)])
    if with_deg:
        pltpu.sync_copy(deg_sh.at[pl.ds(s * RPT, RPT)],
                        deg_out.at[c, pl.ds(s * RPT, RPT)])


@functools.lru_cache(maxsize=None)
def _make_sc(with_deg):
    out_type = [jax.ShapeDtypeStruct((NC, NPAD, D), _bf16)]
    if with_deg:
        out_type.append(jax.ShapeDtypeStruct((NC, NPAD, DEGW), _f32))
    scratch = [
        pltpu.VMEM((NCHUNK, CH), jnp.int32),
        pltpu.VMEM((NCHUNK, CH), jnp.int32),
        pltpu.VMEM((CH, D), _bf16),
        pltpu.VMEM((CH, D), _bf16),
        pltpu.VMEM((CH, D), _bf16),
        pltpu.VMEM((CH, D), _bf16),
        pltpu.VMEM((CH, D), _bf16),
        pltpu.VMEM((128, D), _bf16),
    ]
    if with_deg:
        scratch += [
            pltpu.VMEM((CH, DEGW), _f32),
            pltpu.VMEM((128, DEGW), _f32),
        ]
    scratch.append(pltpu.VMEM_SHARED((NPAD, D), _bf16))
    if with_deg:
        scratch.append(pltpu.VMEM_SHARED((NPAD, DEGW), _f32))
    scratch += [pltpu.SemaphoreType.DMA] * (11 if with_deg else 10)
    mesh = plsc.VectorSubcoreMesh(core_axis_name="c", subcore_axis_name="s")
    return pl.kernel(
        functools.partial(_sc_body, with_deg),
        out_type=out_type,
        mesh=mesh,
        scratch_types=scratch,
        compiler_params=pltpu.CompilerParams(use_tc_tiling_on_sc=False),
        name="sage_sc_agg_deg" if with_deg else "sage_sc_agg",
    )


_BM = 2048
_GRID = (N + _BM - 1) // _BM

_dot = functools.partial(jnp.dot, preferred_element_type=_f32,
                         precision=jax.lax.Precision.HIGHEST)


def _proj_body(x_ref, w_ref, p_ref):
    p_ref[...] = _dot(x_ref[...], w_ref[...]).astype(p_ref.dtype)


def _tc2_body(s1_ref, agg_ref, deg_ref, b_ref, wn_ref, h_ref, p_ref):
    deg = jnp.maximum(deg_ref[0, :, 0:1] + deg_ref[1, :, 0:1], 1.0)
    agg = agg_ref[0].astype(_f32) + agg_ref[1].astype(_f32)
    h = jnp.maximum(s1_ref[...] + agg / deg + b_ref[...], 0.0)
    h_ref[...] = h
    p_ref[...] = _dot(h, wn_ref[...]).astype(_bf16)


def _tc3_body(s2_ref, agg_ref, deg_ref, b_ref, o_ref):
    deg = jnp.maximum(deg_ref[0, :, 0:1] + deg_ref[1, :, 0:1], 1.0)
    agg = agg_ref[0].astype(_f32) + agg_ref[1].astype(_f32)
    o_ref[...] = s2_ref[...] + agg / deg + b_ref[...]


_row_spec = pl.BlockSpec((_BM, D), lambda i: (i, 0))
_w_spec = pl.BlockSpec((D, D), lambda i: (0, 0))
_b_spec = pl.BlockSpec((1, D), lambda i: (0, 0))
_agg_spec = pl.BlockSpec((NC, _BM, D), lambda i: (0, i, 0))
_deg_spec = pl.BlockSpec((NC, _BM, DEGW), lambda i: (0, i, 0))
_row_out = jax.ShapeDtypeStruct((N, D), _f32)
_p_out = jax.ShapeDtypeStruct((N, D), _bf16)

_proj_bf16 = pl.pallas_call(
    _proj_body,
    grid=(_GRID,),
    in_specs=[_row_spec, _w_spec],
    out_specs=_row_spec,
    out_shape=_p_out,
)

_proj_f32 = pl.pallas_call(
    _proj_body,
    grid=(_GRID,),
    in_specs=[_row_spec, _w_spec],
    out_specs=_row_spec,
    out_shape=_row_out,
)

_tc2 = pl.pallas_call(
    _tc2_body,
    grid=(_GRID,),
    in_specs=[_row_spec, _agg_spec, _deg_spec, _b_spec, _w_spec],
    out_specs=[_row_spec, _row_spec],
    out_shape=[_row_out, _p_out],
)

_tc3 = pl.pallas_call(
    _tc3_body,
    grid=(_GRID,),
    in_specs=[_row_spec, _agg_spec, _deg_spec, _b_spec],
    out_specs=_row_spec,
    out_shape=_row_out,
)


def kernel(x, edge_index, W1_self, W1_neigh, b1, W2_self, W2_neigh, b2):
    ei = edge_index.reshape(2, NW, NCHUNK, CH)
    b1r = b1.reshape(1, D)
    b2r = b2.reshape(1, D)

    p1 = _proj_bf16(x, W1_neigh)
    agg1, deg = _make_sc(True)(p1, ei)
    s1 = _proj_f32(x, W1_self)
    h, p2 = _tc2(s1, agg1, deg, b1r, W2_neigh)
    agg2 = _make_sc(False)(p2, ei)[0]
    s2 = _proj_f32(h, W2_self)
    out = _tc3(s2, agg2, deg, b2r)
    return out
